# baseline (speedup 1.0000x reference)
import jax
import jax.numpy as jnp
from jax import lax
from jax.experimental import pallas as pl
from jax.experimental.pallas import tpu as pltpu
from jax.experimental.pallas import tpu_sc as plsc

N_NODES = 10000
N_INC = 320000
D = 128
NC = 2
NS = 16
L = 16
CK = 80
NROWS = N_INC // CK
PADROWS = 4480
SLAB = 256
NH = 5120
N_PAD = NC * NH
R_ACC = 5248
CSHARE = R_ACC // NS
NWR = CSHARE // CK
CTAIL = CSHARE - NWR * CK
PAD_ID = 2 * N_PAD
INFLIGHT = 4
CKZ = 80


def _sc_body(table, he2, seg2, pvec, z128, o128, out,
             he_v, seg_v, rows_v, cbuf, idx_z, p_v, acc, gsem):
    c = lax.axis_index("c")
    s = lax.axis_index("s")

    pltpu.sync_copy(pvec, p_v)
    pltpu.sync_copy(z128, rows_v)
    pltpu.sync_copy(o128, cbuf)

    p = p_v[...][0]
    prow_up = (p + CK - 1) // CK
    rs_c = jnp.where(c == 0, 0, (p // (CK * 8)) * 8)
    span = jnp.where(c == 0, prow_up, NROWS - rs_c)
    cpt = ((span + NS * 8 - 1) // (NS * 8)) * 8
    rowstart = pl.multiple_of(rs_c + s * cpt, 8)

    pltpu.sync_copy(he2.at[pl.ds(rowstart, SLAB)], he_v)
    pltpu.sync_copy(seg2.at[pl.ds(rowstart, SLAB)], seg_v)

    lo = c * NH

    @pl.loop(0, SLAB)
    def _remap(j):
        for m in range(CK // L):
            v = seg_v[j, pl.ds(m * L, L)]
            t = v - lo
            bad = jnp.logical_or(t < 0, t >= NH)
            seg_v[j, pl.ds(m * L, L)] = jnp.where(bad, jnp.int32(NH), t)

    base = s * CSHARE
    iota = lax.iota(jnp.int32, L)
    for r in range(5):
        for m in range(CKZ // L):
            pos = r * CKZ + m * L + iota
            idx_z[r, pl.ds(m * L, L)] = jnp.where(pos < CSHARE, base + pos,
                                                  jnp.int32(NH))

    @pl.loop(0, 5)
    def _zinit(r):
        pltpu.sync_copy(rows_v, acc.at[idx_z.at[r]])
    plsc.subcore_barrier()

    @pl.loop(0, cpt)
    def _cchunk(j):
        pltpu.sync_copy(cbuf, acc.at[seg_v.at[j]], add=True)

    plsc.subcore_barrier()

    @pl.loop(0, NWR)
    def _wrcnt(r):
        off = pl.multiple_of(base + r * CK, 8)
        pltpu.sync_copy(acc.at[pl.ds(off, CK)], cbuf)
        pltpu.sync_copy(cbuf, out.at[1, c, pl.ds(off, CK)])
    offt = pl.multiple_of(base + NWR * CK, 8)
    pltpu.sync_copy(acc.at[pl.ds(offt, CTAIL)], cbuf.at[pl.ds(0, CTAIL)])
    pltpu.sync_copy(cbuf.at[pl.ds(0, CTAIL)], out.at[1, c, pl.ds(offt, CTAIL)])
    plsc.subcore_barrier()

    @pl.loop(0, cpt)
    def _schunk(j):
        pltpu.async_copy(table.at[he_v.at[j]], rows_v, gsem).wait()
        pltpu.sync_copy(rows_v, acc.at[seg_v.at[j]], add=True)

    plsc.subcore_barrier()

    @pl.loop(0, NWR)
    def _wrsum(r):
        off = pl.multiple_of(base + r * CK, 8)
        pltpu.sync_copy(acc.at[pl.ds(off, CK)], rows_v)
        pltpu.sync_copy(rows_v, out.at[0, c, pl.ds(off, CK)])
    pltpu.sync_copy(acc.at[pl.ds(offt, CTAIL)], rows_v.at[pl.ds(0, CTAIL)])
    pltpu.sync_copy(rows_v.at[pl.ds(0, CTAIL)], out.at[0, c, pl.ds(offt, CTAIL)])


def _combine_body(ps_ref, pc_ref, o_ref):
    cnt = pc_ref[0, 0]
    o_ref[...] = (ps_ref[0, 0] - cnt) / jnp.maximum(cnt[:, 0:1], 1.0)


@jax.jit
def kernel(embedding_table, hyperedge_ids, segment_ids):
    npad = PADROWS * CK - N_INC
    he2 = jnp.concatenate(
        [hyperedge_ids, jnp.zeros((npad,), jnp.int32)]).reshape(PADROWS, CK)
    seg2 = jnp.concatenate(
        [segment_ids, jnp.full((npad,), PAD_ID, jnp.int32)]).reshape(PADROWS, CK)
    p = jnp.searchsorted(segment_ids, jnp.int32(NH)).astype(jnp.int32)
    pvec = jnp.full((16,), p, jnp.int32)
    z128 = jnp.zeros((CK, D), jnp.float32)
    o128 = jnp.ones((CK, D), jnp.float32)

    mesh = plsc.VectorSubcoreMesh(core_axis_name="c", subcore_axis_name="s",
                                  num_cores=NC, num_subcores=NS)
    sc_out = pl.kernel(
        _sc_body,
        out_type=jax.ShapeDtypeStruct((2, NC, R_ACC, D), jnp.float32),
        mesh=mesh,
        scratch_types=(
            pltpu.VMEM((SLAB, CK), jnp.int32),
            pltpu.VMEM((SLAB, CK), jnp.int32),
            pltpu.VMEM((CK, D), jnp.float32),
            pltpu.VMEM((CK, D), jnp.float32),
            pltpu.VMEM((5, CKZ), jnp.int32),
            pltpu.VMEM((16,), jnp.int32),
            pltpu.VMEM_SHARED((R_ACC, D), jnp.float32),
            pltpu.SemaphoreType.DMA,
        ),
    )(embedding_table, he2, seg2, pvec, z128, o128)

    rb = 1024
    nb_sc = NH // rb
    out = pl.pallas_call(
        _combine_body,
        grid=(N_PAD // rb,),
        in_specs=[
            pl.BlockSpec((1, 1, rb, D), lambda i: (0, i // nb_sc, i % nb_sc, 0)),
            pl.BlockSpec((1, 1, rb, D), lambda i: (1, i // nb_sc, i % nb_sc, 0)),
        ],
        out_specs=pl.BlockSpec((rb, D), lambda i: (i, 0)),
        out_shape=jax.ShapeDtypeStruct((N_PAD, D), jnp.float32),
    )(sc_out, sc_out)
    return out[:N_NODES]


# speedup vs baseline: 1.0450x; 1.0450x over previous
import jax
import jax.numpy as jnp
from jax import lax
from jax.experimental import pallas as pl
from jax.experimental.pallas import tpu as pltpu
from jax.experimental.pallas import tpu_sc as plsc

N_NODES = 10000
N_INC = 320000
D = 128
NC = 2
NS = 16
L = 16
CK = 80
NROWS = N_INC // CK
PADROWS = 4480
SLAB = 256
NH = 5120
N_PAD = NC * NH
R_ACC = 5248
CSHARE = R_ACC // NS
NWR = CSHARE // CK
CTAIL = CSHARE - NWR * CK
PAD_ID = 2 * N_PAD
INFLIGHT = 4
CKZ = 80


def _sc_body(table, he2, seg2, pvec, z128, o128, out,
             he_v, seg_v, rows_v, cbuf, idx_z, p_v, acc, gsem):
    c = lax.axis_index("c")
    s = lax.axis_index("s")

    c = 1 - c
    pltpu.sync_copy(pvec, p_v)
    pltpu.sync_copy(z128, rows_v)
    pltpu.sync_copy(o128, cbuf)

    p = p_v[...][0]
    prow_up = (p + CK - 1) // CK
    rs_c = jnp.where(c == 0, 0, (p // (CK * 8)) * 8)
    span = jnp.where(c == 0, prow_up, NROWS - rs_c)
    cpt = ((span + NS * 8 - 1) // (NS * 8)) * 8
    rowstart = pl.multiple_of(rs_c + s * cpt, 8)

    pltpu.sync_copy(he2.at[pl.ds(rowstart, SLAB)], he_v)
    pltpu.sync_copy(seg2.at[pl.ds(rowstart, SLAB)], seg_v)

    lo = c * NH

    @pl.loop(0, SLAB)
    def _remap(j):
        for m in range(CK // L):
            v = seg_v[j, pl.ds(m * L, L)]
            t = v - lo
            bad = jnp.logical_or(t < 0, t >= NH)
            trash = NH + ((lax.iota(jnp.int32, L) + m * L) & 63)
            seg_v[j, pl.ds(m * L, L)] = jnp.where(bad, trash, t)

    base = s * CSHARE
    iota = lax.iota(jnp.int32, L)
    for r in range(5):
        for m in range(CKZ // L):
            pos = r * CKZ + m * L + iota
            idx_z[r, pl.ds(m * L, L)] = jnp.where(pos < CSHARE, base + pos,
                                                  jnp.int32(NH))

    @pl.loop(0, 5)
    def _zinit(r):
        pltpu.sync_copy(rows_v, acc.at[idx_z.at[r]])
    plsc.subcore_barrier()

    @pl.loop(0, cpt)
    def _cchunk(j):
        pltpu.sync_copy(cbuf, acc.at[seg_v.at[j]], add=True)

    plsc.subcore_barrier()

    @pl.loop(0, NWR)
    def _wrcnt(r):
        off = pl.multiple_of(base + r * CK, 8)
        pltpu.sync_copy(acc.at[pl.ds(off, CK)], cbuf)
        pltpu.sync_copy(cbuf, out.at[1, c, pl.ds(off, CK)])
    offt = pl.multiple_of(base + NWR * CK, 8)
    pltpu.sync_copy(acc.at[pl.ds(offt, CTAIL)], cbuf.at[pl.ds(0, CTAIL)])
    pltpu.sync_copy(cbuf.at[pl.ds(0, CTAIL)], out.at[1, c, pl.ds(offt, CTAIL)])
    plsc.subcore_barrier()

    @pl.loop(0, cpt)
    def _schunk(j):
        pltpu.async_copy(table.at[he_v.at[j]], rows_v, gsem).wait()
        pltpu.sync_copy(rows_v, acc.at[seg_v.at[j]], add=True)

    plsc.subcore_barrier()

    @pl.loop(0, NWR)
    def _wrsum(r):
        off = pl.multiple_of(base + r * CK, 8)
        pltpu.sync_copy(acc.at[pl.ds(off, CK)], rows_v)
        pltpu.sync_copy(rows_v, out.at[0, c, pl.ds(off, CK)])
    pltpu.sync_copy(acc.at[pl.ds(offt, CTAIL)], rows_v.at[pl.ds(0, CTAIL)])
    pltpu.sync_copy(rows_v.at[pl.ds(0, CTAIL)], out.at[0, c, pl.ds(offt, CTAIL)])


def _combine_body(ps_ref, pc_ref, o_ref):
    cnt = pc_ref[0, 0]
    o_ref[...] = (ps_ref[0, 0] - cnt) / jnp.maximum(cnt[:, 0:1], 1.0)


@jax.jit
def kernel(embedding_table, hyperedge_ids, segment_ids):
    npad = PADROWS * CK - N_INC
    he2 = jnp.concatenate(
        [hyperedge_ids, jnp.zeros((npad,), jnp.int32)]).reshape(PADROWS, CK)
    seg2 = jnp.concatenate(
        [segment_ids, jnp.full((npad,), PAD_ID, jnp.int32)]).reshape(PADROWS, CK)
    p = jnp.searchsorted(segment_ids, jnp.int32(NH)).astype(jnp.int32)
    pvec = jnp.full((16,), p, jnp.int32)
    z128 = jnp.zeros((CK, D), jnp.float32)
    o128 = jnp.ones((CK, D), jnp.float32)

    mesh = plsc.VectorSubcoreMesh(core_axis_name="c", subcore_axis_name="s",
                                  num_cores=NC, num_subcores=NS)
    sc_out = pl.kernel(
        _sc_body,
        out_type=jax.ShapeDtypeStruct((2, NC, R_ACC, D), jnp.float32),
        mesh=mesh,
        scratch_types=(
            pltpu.VMEM((SLAB, CK), jnp.int32),
            pltpu.VMEM((SLAB, CK), jnp.int32),
            pltpu.VMEM((CK, D), jnp.float32),
            pltpu.VMEM((CK, D), jnp.float32),
            pltpu.VMEM((5, CKZ), jnp.int32),
            pltpu.VMEM((16,), jnp.int32),
            pltpu.VMEM_SHARED((R_ACC, D), jnp.float32),
            pltpu.SemaphoreType.DMA,
        ),
    )(embedding_table, he2, seg2, pvec, z128, o128)

    rb = 1024
    nb_sc = NH // rb
    out = pl.pallas_call(
        _combine_body,
        grid=(N_PAD // rb,),
        in_specs=[
            pl.BlockSpec((1, 1, rb, D), lambda i: (0, i // nb_sc, i % nb_sc, 0)),
            pl.BlockSpec((1, 1, rb, D), lambda i: (1, i // nb_sc, i % nb_sc, 0)),
        ],
        out_specs=pl.BlockSpec((rb, D), lambda i: (i, 0)),
        out_shape=jax.ShapeDtypeStruct((N_PAD, D), jnp.float32),
    )(sc_out, sc_out)
    return out[:N_NODES]
